# baseline (speedup 1.0000x reference)
import functools

import jax
import jax.numpy as jnp
from jax.experimental import pallas as pl
from jax.experimental.pallas import tpu as pltpu

N = 10000
E = 320000
G = 16
D = 512
L = 8
NT = 51
GT = 11
HID = 32

BN = 1000
GRID_N = N // BN


def _l2norm_rows(y):
    n = jnp.sqrt(jnp.sum(y * y, axis=1, keepdims=True))
    return y / jnp.maximum(n, 1e-12)


def _premp_body(x_ref, w_ref, o_ref):
    y = jax.nn.relu(jnp.dot(x_ref[...], w_ref[...],
                            preferred_element_type=jnp.float32))
    o_ref[...] = _l2norm_rows(y)


def _premp(x128, w128):
    return pl.pallas_call(
        _premp_body,
        grid=(GRID_N,),
        in_specs=[pl.BlockSpec((BN, 128), lambda i: (i, 0)),
                  pl.BlockSpec((128, D), lambda i: (0, 0))],
        out_specs=pl.BlockSpec((BN, D), lambda i: (i, 0)),
        out_shape=jax.ShapeDtypeStruct((N, D), jnp.float32),
    )(x128, w128)


def _proj_body(h_ref, w_ref, o_ref):
    o_ref[...] = jnp.dot(h_ref[...], w_ref[...],
                         preferred_element_type=jnp.float32)


def _proj(h, wcat):
    return pl.pallas_call(
        _proj_body,
        grid=(GRID_N,),
        in_specs=[pl.BlockSpec((BN, D), lambda i: (i, 0)),
                  pl.BlockSpec((D, 4 * D), lambda i: (0, 0))],
        out_specs=pl.BlockSpec((BN, 4 * D), lambda i: (i, 0)),
        out_shape=jax.ShapeDtypeStruct((N, 4 * D), jnp.float32),
    )(h, wcat)


def _epi_body(agg_ref, s_ref, hin_ref, o_ref):
    y = jax.nn.relu(agg_ref[...] + s_ref[...])
    o_ref[...] = _l2norm_rows(y) + hin_ref[...]


def _epilogue(agg, s, h_in):
    return pl.pallas_call(
        _epi_body,
        grid=(GRID_N,),
        in_specs=[pl.BlockSpec((BN, D), lambda i: (i, 0))] * 3,
        out_specs=pl.BlockSpec((BN, D), lambda i: (i, 0)),
        out_shape=jax.ShapeDtypeStruct((N, D), jnp.float32),
    )(agg, s, h_in)


def _heads_body(h_ref, w1_ref, w2b_ref, gmask_ref, b2_ref, batch_ref,
                np_ref, pool_ref):
    i = pl.program_id(0)
    hb = _l2norm_rows(h_ref[...])
    z = jax.nn.relu(jnp.dot(hb, w1_ref[...], preferred_element_type=jnp.float32))
    ss = jnp.dot(z * z, gmask_ref[...], preferred_element_type=jnp.float32)
    denom = jnp.maximum(jnp.sqrt(ss), 1e-12)
    num = jnp.dot(z, w2b_ref[...], preferred_element_type=jnp.float32)
    np_ref[...] = num / denom + b2_ref[...]
    mask = (batch_ref[0] == jax.lax.broadcasted_iota(jnp.int32, (G, BN), 0)
            ).astype(jnp.float32)
    pool = jnp.dot(mask, hb, preferred_element_type=jnp.float32)

    @pl.when(i == 0)
    def _():
        pool_ref[...] = jnp.zeros_like(pool_ref)

    pool_ref[...] += pool


def _heads(h, w1r, w2b, gmask, b2row, batch3):
    return pl.pallas_call(
        _heads_body,
        grid=(GRID_N,),
        in_specs=[pl.BlockSpec((BN, D), lambda i: (i, 0)),
                  pl.BlockSpec((D, NT * HID), lambda i: (0, 0)),
                  pl.BlockSpec((NT * HID, NT), lambda i: (0, 0)),
                  pl.BlockSpec((NT * HID, NT), lambda i: (0, 0)),
                  pl.BlockSpec((1, NT), lambda i: (0, 0)),
                  pl.BlockSpec((1, 1, BN), lambda i: (i, 0, 0))],
        out_specs=[pl.BlockSpec((BN, NT), lambda i: (i, 0)),
                   pl.BlockSpec((G, D), lambda i: (0, 0))],
        out_shape=[jax.ShapeDtypeStruct((N, NT), jnp.float32),
                   jax.ShapeDtypeStruct((G, D), jnp.float32)],
    )(h, w1r, w2b, gmask, b2row, batch3)


def _ghead_body(g_ref, wg1_ref, wg2_ref, bg2_ref, o_ref):
    gh = _l2norm_rows(jax.nn.relu(
        jnp.dot(g_ref[...], wg1_ref[...], preferred_element_type=jnp.float32)))
    o_ref[...] = jnp.dot(gh, wg2_ref[...],
                         preferred_element_type=jnp.float32) + bg2_ref[...]


def _ghead(g, wg1, wg2, bg2row):
    return pl.pallas_call(
        _ghead_body,
        in_specs=[pl.BlockSpec((G, D), lambda: (0, 0)),
                  pl.BlockSpec((D, D), lambda: (0, 0)),
                  pl.BlockSpec((D, GT), lambda: (0, 0)),
                  pl.BlockSpec((1, GT), lambda: (0, 0))],
        out_specs=pl.BlockSpec((G, GT), lambda: (0, 0)),
        out_shape=jax.ShapeDtypeStruct((G, GT), jnp.float32),
    )(g, wg1, wg2, bg2row)


def _edge_stage(k, q, v, src, dst):
    gate = jax.nn.sigmoid(k[dst] + q[src])
    return jax.ops.segment_sum(gate * v[src], dst, num_segments=N)


def kernel(x, edge_index, batch, y, y_graph, Wpre, Wk, Wq, Wv, Ws,
           W1, W2, b2, Wg1, Wg2, bg2):
    src = edge_index[0]
    dst = edge_index[1]

    x128 = jnp.pad(x, ((0, 0), (0, 128 - x.shape[1])))
    w128 = jnp.pad(Wpre, ((0, 128 - Wpre.shape[0]), (0, 0)))
    h = _premp(x128, w128)

    wcat = jnp.concatenate([Wk, Wq, Wv, Ws], axis=2)

    for l in range(L):
        h_in = h
        kqvs = _proj(h, wcat[l])
        k = kqvs[:, 0:D]
        q = kqvs[:, D:2 * D]
        v = kqvs[:, 2 * D:3 * D]
        s = kqvs[:, 3 * D:4 * D]
        agg = _edge_stage(k, q, v, src, dst)
        h = _epilogue(agg, s, h_in)

    w1r = W1.transpose(1, 0, 2).reshape(D, NT * HID)
    hh = jnp.arange(NT)
    kk = jnp.arange(HID)
    w2b = jnp.zeros((NT, HID, NT), jnp.float32).at[
        hh[:, None], kk[None, :], hh[:, None]].set(W2).reshape(NT * HID, NT)
    gmask = jnp.zeros((NT, HID, NT), jnp.float32).at[
        hh[:, None], kk[None, :], hh[:, None]].set(1.0).reshape(NT * HID, NT)
    batch3 = batch.astype(jnp.int32).reshape(GRID_N, 1, BN)
    node_pred, g = _heads(h, w1r, w2b, gmask, b2[None, :], batch3)
    graph_pred = _ghead(g, Wg1, Wg2, bg2[None, :])

    pred = jnp.vstack([jnp.pad(node_pred, ((0, 0), (0, GT))),
                       jnp.pad(graph_pred, ((0, 0), (NT, 0)))])
    true = jnp.vstack([jnp.pad(y, ((0, 0), (0, GT))),
                       jnp.pad(y_graph, ((0, 0), (NT, 0)))])
    return pred, true
